# Initial kernel scaffold; baseline (speedup 1.0000x reference)
import functools

import jax
import jax.numpy as jnp
from jax import lax
from jax.experimental import pallas as pl
from jax.experimental.pallas import tpu as pltpu
from jax.experimental.pallas import tpu_sc as plsc

_PACK_BASE = 100003
_B_LO = 34467
_NC, _NS = 2, 16
_NW = _NC * _NS
_L = 16
_C = 128
_W = 64


def _u32(x):
    return plsc.bitcast(x, jnp.uint32)


def _i32(x):
    return plsc.bitcast(x, jnp.int32)


def _carry(x, y, s):
    return ((x & y) | ((x | y) & ~s)) >> 31


def _pack_limbs(a, b, c):
    a, b, c = _u32(a), _u32(b), _u32(c)
    t = a * _B_LO
    s = t + b
    a_sh = a << 16
    u_lo = a_sh + s
    u_hi = (a >> 16) + _carry(a_sh, s, u_lo)
    p1 = u_lo >> 16
    p0 = u_lo & 0xFFFF
    q = p1 * _B_LO + p0
    q_sh = q << 16
    r = p0 * _B_LO
    x1 = q_sh + r
    c2 = _carry(q_sh, r, x1)
    key_lo = x1 + c
    c3 = _carry(x1, c, key_lo)
    key_hi = u_hi * _PACK_BASE + p1 + (q >> 16) + c2 + c3
    hi31 = _i32((key_hi << 1) | (key_lo >> 31))
    lo31 = _i32(key_lo & 0x7FFFFFFF)
    return hi31, lo31


def _make_sc_search(n_pad, sample, steps_a):
    chunks = n_pad // (_NW * _C)
    mesh = plsc.VectorSubcoreMesh(
        core_axis_name="c", subcore_axis_name="s",
        num_cores=_NC, num_subcores=_NS)

    @functools.partial(
        pl.kernel,
        out_type=jax.ShapeDtypeStruct((n_pad,), jnp.int32),
        mesh=mesh,
        scratch_types=[
            pltpu.VMEM((sample,), jnp.int32),
            pltpu.VMEM((sample,), jnp.int32),
            pltpu.VMEM((_C,), jnp.int32),
            pltpu.VMEM((_C,), jnp.int32),
            pltpu.VMEM((_C,), jnp.int32),
            pltpu.VMEM((_C,), jnp.int32),
            pltpu.VMEM((_C, _W), jnp.int32),
            pltpu.VMEM((_C, _W), jnp.int32),
            pltpu.VMEM((_C,), jnp.int32),
            pltpu.SemaphoreType.DMA,
            pltpu.SemaphoreType.DMA,
        ],
    )
    def sc_search(a_hbm, b_hbm, c_hbm, fhi_hbm, flo_hbm, shi_hbm, slo_hbm,
                  out_hbm, smp_hi, smp_lo, av, bv, cv, bktv, win_hi, win_lo,
                  outv, sem1, sem2):
        wid = lax.axis_index("s") * _NC + lax.axis_index("c")
        pltpu.sync_copy(shi_hbm, smp_hi)
        pltpu.sync_copy(slo_hbm, smp_lo)
        tile_base = wid * (chunks * _C)

        def chunk_body(k, carry):
            base = tile_base + k * _C
            pltpu.sync_copy(a_hbm.at[pl.ds(base, _C)], av)
            pltpu.sync_copy(b_hbm.at[pl.ds(base, _C)], bv)
            pltpu.sync_copy(c_hbm.at[pl.ds(base, _C)], cv)
            keys = []
            for g in range(_C // _L):
                a16 = av[pl.ds(g * _L, _L)]
                b16 = bv[pl.ds(g * _L, _L)]
                c16 = cv[pl.ds(g * _L, _L)]
                qh, ql = _pack_limbs(a16, b16, c16)
                keys.append((qh, ql))
                pos = jnp.zeros((_L,), jnp.int32)
                for sstep in range(steps_a - 1, -1, -1):
                    t = pos + (1 << sstep)
                    th = plsc.load_gather(smp_hi, [t - 1])
                    tl = plsc.load_gather(smp_lo, [t - 1])
                    less = (th < qh) | ((th == qh) & (tl < ql))
                    pos = jnp.where(less, t, pos)
                bktv[pl.ds(g * _L, _L)] = pos
            cp1 = pltpu.async_copy(fhi_hbm.at[bktv], win_hi, sem1)
            cp2 = pltpu.async_copy(flo_hbm.at[bktv], win_lo, sem2)
            cp1.wait()
            cp2.wait()
            for g in range(_C // _L):
                qh, ql = keys[g]
                row = lax.iota(jnp.int32, _L) + (g * _L)
                pos = jnp.zeros((_L,), jnp.int32)
                for sstep in range(5, -1, -1):
                    t = pos + (1 << sstep)
                    th = plsc.load_gather(win_hi, [row, t - 1])
                    tl = plsc.load_gather(win_lo, [row, t - 1])
                    less = (th < qh) | ((th == qh) & (tl < ql))
                    pos = jnp.where(less, t, pos)
                fh = plsc.load_gather(win_hi, [row, pos])
                fl = plsc.load_gather(win_lo, [row, pos])
                hit = (fh == qh) & (fl == ql)
                outv[pl.ds(g * _L, _L)] = hit.astype(jnp.int32)
            pltpu.sync_copy(outv, out_hbm.at[pl.ds(base, _C)])
            return carry

        lax.fori_loop(0, chunks, chunk_body, 0)

    return sc_search


@jax.jit
def _fact_index(atoms, fact_hashes):
    n = atoms.shape[0]
    f = fact_hashes.shape[0]

    f_pad = max(_W, 1 << (f - 1).bit_length())
    sample = f_pad // _W
    steps_a = sample.bit_length() - 1
    fhi = (fact_hashes >> 31).astype(jnp.int32)
    flo = (fact_hashes & 0x7FFFFFFF).astype(jnp.int32)
    fhi = jnp.concatenate(
        [fhi, jnp.full((f_pad - f,), 0x3FFFFFFF, jnp.int32)])
    flo = jnp.concatenate(
        [flo, jnp.full((f_pad - f,), 0x7FFFFFFF, jnp.int32)])
    fhi2d = fhi.reshape(sample, _W)
    flo2d = flo.reshape(sample, _W)
    smp_hi = fhi2d[:, _W - 1]
    smp_lo = flo2d[:, _W - 1]

    per = _NW * _C
    n_pad = ((n + per - 1) // per) * per
    cols = []
    for j in range(3):
        col = atoms[:, j].astype(jnp.int32)
        cols.append(jnp.concatenate(
            [col, jnp.zeros((n_pad - n,), jnp.int32)]))

    sc_search = _make_sc_search(n_pad, sample, steps_a)
    out = sc_search(cols[0], cols[1], cols[2], fhi2d, flo2d, smp_hi, smp_lo)
    return out[:n] != 0


def kernel(atoms, fact_hashes):
    if atoms.shape[0] == 0 or fact_hashes.shape[0] == 0:
        return jnp.zeros((atoms.shape[0],), dtype=bool)
    return _fact_index(atoms, fact_hashes)


# speedup vs baseline: 10.9866x; 10.9866x over previous
import functools

import jax
import jax.numpy as jnp
from jax import lax
from jax.experimental import pallas as pl
from jax.experimental.pallas import tpu as pltpu
from jax.experimental.pallas import tpu_sc as plsc

_PACK_BASE = 100003
_B_LO = 34467
_NC, _NS = 2, 16
_NW = _NC * _NS
_L = 16
_C = 128
_W = 64


def _u32(x):
    return plsc.bitcast(x, jnp.uint32)


def _i32(x):
    return plsc.bitcast(x, jnp.int32)


def _carry(x, y, s):
    return ((x & y) | ((x | y) & ~s)) >> 31


def _pack_limbs(a, b, c):
    a, b, c = _u32(a), _u32(b), _u32(c)
    t = a * _B_LO
    s = t + b
    a_sh = a << 16
    u_lo = a_sh + s
    u_hi = (a >> 16) + _carry(a_sh, s, u_lo)
    p1 = u_lo >> 16
    p0 = u_lo & 0xFFFF
    q = p1 * _B_LO + p0
    q_sh = q << 16
    r = p0 * _B_LO
    x1 = q_sh + r
    c2 = _carry(q_sh, r, x1)
    key_lo = x1 + c
    c3 = _carry(x1, c, key_lo)
    key_hi = u_hi * _PACK_BASE + p1 + (q >> 16) + c2 + c3
    hi31 = _i32((key_hi << 1) | (key_lo >> 31))
    lo31 = _i32(key_lo & 0x7FFFFFFF)
    return hi31, lo31


def _make_sc_search(n_pad, sample, steps_a):
    chunks = n_pad // (_NW * _C)
    mesh = plsc.VectorSubcoreMesh(
        core_axis_name="c", subcore_axis_name="s",
        num_cores=_NC, num_subcores=_NS)

    @functools.partial(
        pl.kernel,
        out_type=jax.ShapeDtypeStruct((n_pad,), jnp.int32),
        mesh=mesh,
        scratch_types=[
            pltpu.VMEM((sample,), jnp.int32),
            pltpu.VMEM((sample,), jnp.int32),
            pltpu.VMEM((_C,), jnp.int32),
            pltpu.VMEM((_C,), jnp.int32),
            pltpu.VMEM((_C,), jnp.int32),
            pltpu.VMEM((_C,), jnp.int32),
            pltpu.VMEM((_C, 2 * _W), jnp.int32),
            pltpu.VMEM((_C,), jnp.int32),
            pltpu.SemaphoreType.DMA,
        ],
        compiler_params=pltpu.CompilerParams(needs_layout_passes=False),
    )
    def sc_search(a_hbm, b_hbm, c_hbm, fcomb_hbm, shi_hbm, slo_hbm,
                  out_hbm, smp_hi, smp_lo, av, bv, cv, bktv, win,
                  outv, sem1):
        wid = lax.axis_index("s") * _NC + lax.axis_index("c")
        pltpu.sync_copy(shi_hbm, smp_hi)
        pltpu.sync_copy(slo_hbm, smp_lo)
        tile_base = wid * jnp.int32(chunks * _C)

        def chunk_body(k, carry):
            base = tile_base + k * jnp.int32(_C)
            pltpu.sync_copy(a_hbm.at[pl.ds(base, _C)], av)
            pltpu.sync_copy(b_hbm.at[pl.ds(base, _C)], bv)
            pltpu.sync_copy(c_hbm.at[pl.ds(base, _C)], cv)
            keys = []
            for g in range(_C // _L):
                a16 = av[pl.ds(g * _L, _L)]
                b16 = bv[pl.ds(g * _L, _L)]
                c16 = cv[pl.ds(g * _L, _L)]
                qh, ql = _pack_limbs(a16, b16, c16)
                keys.append((qh, ql))
                pos = jnp.zeros((_L,), jnp.int32)
                for sstep in range(steps_a - 1, -1, -1):
                    t = pos + (1 << sstep)
                    th = plsc.load_gather(smp_hi, [t - 1])
                    tl = plsc.load_gather(smp_lo, [t - 1])
                    less = (th < qh) | ((th == qh) & (tl < ql))
                    pos = jnp.where(less, t, pos)
                bktv[pl.ds(g * _L, _L)] = pos
            pltpu.async_copy(fcomb_hbm.at[bktv], win, sem1).wait()
            for g in range(_C // _L):
                qh, ql = keys[g]
                row = lax.iota(jnp.int32, _L) + (g * _L)
                pos = jnp.zeros((_L,), jnp.int32)
                for sstep in range(5, -1, -1):
                    t = pos + (1 << sstep)
                    th = plsc.load_gather(win, [row, t - 1])
                    tl = plsc.load_gather(win, [row, t + (_W - 1)])
                    less = (th < qh) | ((th == qh) & (tl < ql))
                    pos = jnp.where(less, t, pos)
                fh = plsc.load_gather(win, [row, pos])
                fl = plsc.load_gather(win, [row, pos + _W])
                hit = (fh == qh) & (fl == ql)
                outv[pl.ds(g * _L, _L)] = hit.astype(jnp.int32)
            pltpu.sync_copy(outv, out_hbm.at[pl.ds(base, _C)])
            return carry

        lax.fori_loop(jnp.int32(0), jnp.int32(chunks), chunk_body,
                      jnp.int32(0))

    return sc_search


@jax.jit
def _fact_index(atoms, fact_hashes):
    n = atoms.shape[0]
    f = fact_hashes.shape[0]

    f_pad = max(_W, 1 << (f - 1).bit_length())
    sample = f_pad // _W
    steps_a = sample.bit_length() - 1
    fhi = (fact_hashes >> 31).astype(jnp.int32)
    flo = (fact_hashes & 0x7FFFFFFF).astype(jnp.int32)
    fhi = jnp.concatenate(
        [fhi, jnp.full((f_pad - f,), 0x3FFFFFFF, jnp.int32)])
    flo = jnp.concatenate(
        [flo, jnp.full((f_pad - f,), 0x7FFFFFFF, jnp.int32)])
    fhi2d = fhi.reshape(sample, _W)
    flo2d = flo.reshape(sample, _W)
    fcomb = jnp.concatenate([fhi2d, flo2d], axis=1)
    smp_hi = fhi2d[:, _W - 1]
    smp_lo = flo2d[:, _W - 1]

    per = _NW * _C
    n_pad = ((n + per - 1) // per) * per
    cols = []
    for j in range(3):
        col = atoms[:, j].astype(jnp.int32)
        cols.append(jnp.concatenate(
            [col, jnp.zeros((n_pad - n,), jnp.int32)]))

    sc_search = _make_sc_search(n_pad, sample, steps_a)
    out = sc_search(cols[0], cols[1], cols[2], fcomb, smp_hi, smp_lo)
    return out[:n] != 0


def kernel(atoms, fact_hashes):
    if atoms.shape[0] == 0 or fact_hashes.shape[0] == 0:
        return jnp.zeros((atoms.shape[0],), dtype=bool)
    return _fact_index(atoms, fact_hashes)


# speedup vs baseline: 15.5853x; 1.4186x over previous
import functools

import jax
import jax.numpy as jnp
from jax import lax
from jax.experimental import pallas as pl
from jax.experimental.pallas import tpu as pltpu
from jax.experimental.pallas import tpu_sc as plsc

_PACK_BASE = 100003
_B_LO = 34467
_NC, _NS = 2, 16
_NW = _NC * _NS
_L = 16
_C = 128
_W = 64


def _u32(x):
    return plsc.bitcast(x, jnp.uint32)


def _i32(x):
    return plsc.bitcast(x, jnp.int32)


def _carry(x, y, s):
    return ((x & y) | ((x | y) & ~s)) >> 31


def _pack_limbs(a, b, c):
    a, b, c = _u32(a), _u32(b), _u32(c)
    t = a * _B_LO
    s = t + b
    a_sh = a << 16
    u_lo = a_sh + s
    u_hi = (a >> 16) + _carry(a_sh, s, u_lo)
    p1 = u_lo >> 16
    p0 = u_lo & 0xFFFF
    q = p1 * _B_LO + p0
    q_sh = q << 16
    r = p0 * _B_LO
    x1 = q_sh + r
    c2 = _carry(q_sh, r, x1)
    key_lo = x1 + c
    c3 = _carry(x1, c, key_lo)
    key_hi = u_hi * _PACK_BASE + p1 + (q >> 16) + c2 + c3
    hi31 = _i32((key_hi << 1) | (key_lo >> 31))
    lo31 = _i32(key_lo & 0x7FFFFFFF)
    return hi31, lo31


def _make_sc_search(n_pad, sample, steps_a):
    chunks = n_pad // (_NW * _C)
    assert chunks % 2 == 1
    mesh = plsc.VectorSubcoreMesh(
        core_axis_name="c", subcore_axis_name="s",
        num_cores=_NC, num_subcores=_NS)

    @functools.partial(
        pl.kernel,
        out_type=jax.ShapeDtypeStruct((n_pad,), jnp.int32),
        mesh=mesh,
        scratch_types=[
            pltpu.VMEM((sample,), jnp.int32),
            pltpu.VMEM((sample,), jnp.int32),
            pltpu.VMEM((_C,), jnp.int32),
            pltpu.VMEM((_C,), jnp.int32),
            pltpu.VMEM((_C,), jnp.int32),
            pltpu.VMEM((2, _C), jnp.int32),
            pltpu.VMEM((2, _C), jnp.int32),
            pltpu.VMEM((2, _C), jnp.int32),
            pltpu.VMEM((2, _C, 2 * _W), jnp.int32),
            pltpu.VMEM((_C,), jnp.int32),
            pltpu.SemaphoreType.DMA,
            pltpu.SemaphoreType.DMA,
        ],
        compiler_params=pltpu.CompilerParams(needs_layout_passes=False),
    )
    def sc_search(a_hbm, b_hbm, c_hbm, fcomb_hbm, shi_hbm, slo_hbm,
                  out_hbm, smp_hi, smp_lo, av, bv, cv, bktv, qhv, qlv,
                  win, outv, sem0, sem1):
        wid = lax.axis_index("s") * _NC + lax.axis_index("c")
        pltpu.sync_copy(shi_hbm, smp_hi)
        pltpu.sync_copy(slo_hbm, smp_lo)
        tile_base = wid * jnp.int32(chunks * _C)
        sems = (sem0, sem1)
        ngrp = _C // _L

        def phase_a(base, buf):
            buf = jnp.int32(buf)
            pltpu.sync_copy(a_hbm.at[pl.ds(base, _C)], av)
            pltpu.sync_copy(b_hbm.at[pl.ds(base, _C)], bv)
            pltpu.sync_copy(c_hbm.at[pl.ds(base, _C)], cv)
            for blk in range(0, ngrp, 4):
                keys, pos = [], []
                for g in range(blk, blk + 4):
                    a16 = av[pl.ds(g * _L, _L)]
                    b16 = bv[pl.ds(g * _L, _L)]
                    c16 = cv[pl.ds(g * _L, _L)]
                    qh, ql = _pack_limbs(a16, b16, c16)
                    qhv[buf, pl.ds(g * _L, _L)] = qh
                    qlv[buf, pl.ds(g * _L, _L)] = ql
                    keys.append((qh, ql))
                    pos.append(jnp.zeros((_L,), jnp.int32))
                for sstep in range(steps_a - 1, -1, -1):
                    ts, ths, tls = [], [], []
                    for i in range(4):
                        t = pos[i] + (1 << sstep)
                        ts.append(t)
                        ths.append(plsc.load_gather(smp_hi, [t - 1]))
                        tls.append(plsc.load_gather(smp_lo, [t - 1]))
                    for i in range(4):
                        qh, ql = keys[i]
                        less = (ths[i] < qh) | ((ths[i] == qh) & (tls[i] < ql))
                        pos[i] = jnp.where(less, ts[i], pos[i])
                for i in range(4):
                    bktv[buf, pl.ds((blk + i) * _L, _L)] = pos[i]

        def fire(buf, k):
            return pltpu.async_copy(
                fcomb_hbm.at[bktv.at[jnp.int32(buf)]],
                win.at[jnp.int32(buf)], sems[k % 2])

        def phase_b(base, buf):
            buf = jnp.int32(buf)
            for blk in range(0, ngrp, 4):
                keys, pos, rows = [], [], []
                for g in range(blk, blk + 4):
                    qh = qhv[buf, pl.ds(g * _L, _L)]
                    ql = qlv[buf, pl.ds(g * _L, _L)]
                    keys.append((qh, ql))
                    rows.append(lax.iota(jnp.int32, _L) + (g * _L))
                    pos.append(jnp.zeros((_L,), jnp.int32))
                wref = win.at[buf]
                for sstep in range(5, -1, -1):
                    ts, ths, tls = [], [], []
                    for i in range(4):
                        t = pos[i] + (1 << sstep)
                        ts.append(t)
                        ths.append(plsc.load_gather(wref, [rows[i], t - 1]))
                        tls.append(plsc.load_gather(wref, [rows[i], t + (_W - 1)]))
                    for i in range(4):
                        qh, ql = keys[i]
                        less = (ths[i] < qh) | ((ths[i] == qh) & (tls[i] < ql))
                        pos[i] = jnp.where(less, ts[i], pos[i])
                for i in range(4):
                    qh, ql = keys[i]
                    fh = plsc.load_gather(wref, [rows[i], pos[i]])
                    fl = plsc.load_gather(wref, [rows[i], pos[i] + _W])
                    hit = (fh == qh) & (fl == ql)
                    outv[pl.ds((blk + i) * _L, _L)] = hit.astype(jnp.int32)
            pltpu.sync_copy(outv, out_hbm.at[pl.ds(base, _C)])

        phase_a(tile_base, 0)
        cp = fire(0, 0)

        def pair_body(j, carry):
            k = j * jnp.int32(2)
            base_k = tile_base + k * jnp.int32(_C)
            phase_a(base_k + jnp.int32(_C), 1)
            fire(1, 1)
            pltpu.make_async_copy(fcomb_hbm.at[bktv.at[jnp.int32(0)]],
                                  win.at[jnp.int32(0)], sem0).wait()
            phase_b(base_k, 0)
            phase_a(base_k + jnp.int32(2 * _C), 0)
            fire(0, 0)
            pltpu.make_async_copy(fcomb_hbm.at[bktv.at[jnp.int32(1)]],
                                  win.at[jnp.int32(1)], sem1).wait()
            phase_b(base_k + jnp.int32(_C), 1)
            return carry

        lax.fori_loop(jnp.int32(0), jnp.int32((chunks - 1) // 2), pair_body,
                      jnp.int32(0))
        last = tile_base + jnp.int32((chunks - 1) * _C)
        pltpu.make_async_copy(fcomb_hbm.at[bktv.at[jnp.int32(0)]],
                              win.at[jnp.int32(0)], sem0).wait()
        phase_b(last, 0)

    return sc_search


@jax.jit
def _fact_index(atoms, fact_hashes):
    n = atoms.shape[0]
    f = fact_hashes.shape[0]

    f_pad = max(_W, 1 << (f - 1).bit_length())
    sample = f_pad // _W
    steps_a = sample.bit_length() - 1
    fhi = (fact_hashes >> 31).astype(jnp.int32)
    flo = (fact_hashes & 0x7FFFFFFF).astype(jnp.int32)
    fhi = jnp.concatenate(
        [fhi, jnp.full((f_pad - f,), 0x3FFFFFFF, jnp.int32)])
    flo = jnp.concatenate(
        [flo, jnp.full((f_pad - f,), 0x7FFFFFFF, jnp.int32)])
    fhi2d = fhi.reshape(sample, _W)
    flo2d = flo.reshape(sample, _W)
    fcomb = jnp.concatenate([fhi2d, flo2d], axis=1)
    smp_hi = fhi2d[:, _W - 1]
    smp_lo = flo2d[:, _W - 1]

    per = _NW * _C
    n_chunks = (n + per - 1) // per
    if n_chunks % 2 == 0:
        n_chunks += 1
    n_pad = n_chunks * per
    cols = []
    for j in range(3):
        col = atoms[:, j].astype(jnp.int32)
        cols.append(jnp.concatenate(
            [col, jnp.zeros((n_pad - n,), jnp.int32)]))

    sc_search = _make_sc_search(n_pad, sample, steps_a)
    out = sc_search(cols[0], cols[1], cols[2], fcomb, smp_hi, smp_lo)
    return out[:n] != 0


def kernel(atoms, fact_hashes):
    if atoms.shape[0] == 0 or fact_hashes.shape[0] == 0:
        return jnp.zeros((atoms.shape[0],), dtype=bool)
    return _fact_index(atoms, fact_hashes)


# speedup vs baseline: 15.9810x; 1.0254x over previous
import functools

import jax
import jax.numpy as jnp
from jax import lax
from jax.experimental import pallas as pl
from jax.experimental.pallas import tpu as pltpu
from jax.experimental.pallas import tpu_sc as plsc

_PACK_BASE = 100003
_B_LO = 34467
_NC, _NS = 2, 16
_NW = _NC * _NS
_L = 16
_C = 128
_W = 64
_BLK = 8


def _u32(x):
    return plsc.bitcast(x, jnp.uint32)


def _i32(x):
    return plsc.bitcast(x, jnp.int32)


def _carry(x, y, s):
    return ((x & y) | ((x | y) & ~s)) >> 31


def _pack_limbs(a, b, c):
    a, b, c = _u32(a), _u32(b), _u32(c)
    t = a * _B_LO
    s = t + b
    a_sh = a << 16
    u_lo = a_sh + s
    u_hi = (a >> 16) + _carry(a_sh, s, u_lo)
    p1 = u_lo >> 16
    p0 = u_lo & 0xFFFF
    q = p1 * _B_LO + p0
    q_sh = q << 16
    r = p0 * _B_LO
    x1 = q_sh + r
    c2 = _carry(q_sh, r, x1)
    key_lo = x1 + c
    c3 = _carry(x1, c, key_lo)
    key_hi = u_hi * _PACK_BASE + p1 + (q >> 16) + c2 + c3
    hi31 = _i32((key_hi << 1) | (key_lo >> 31))
    lo31 = _i32(key_lo & 0x7FFFFFFF)
    return hi31, lo31


def _make_sc_search(n_pad, sample, steps_a):
    chunks = n_pad // (_NW * _C)
    assert chunks % 2 == 1
    mesh = plsc.VectorSubcoreMesh(
        core_axis_name="c", subcore_axis_name="s",
        num_cores=_NC, num_subcores=_NS)

    @functools.partial(
        pl.kernel,
        out_type=jax.ShapeDtypeStruct((n_pad,), jnp.int32),
        mesh=mesh,
        scratch_types=[
            pltpu.VMEM((sample,), jnp.int32),
            pltpu.VMEM((sample,), jnp.int32),
            pltpu.VMEM((_C,), jnp.int32),
            pltpu.VMEM((_C,), jnp.int32),
            pltpu.VMEM((_C,), jnp.int32),
            pltpu.VMEM((2, _C), jnp.int32),
            pltpu.VMEM((2, _C), jnp.int32),
            pltpu.VMEM((2, _C), jnp.int32),
            pltpu.VMEM((2, _C, 2 * _W), jnp.int32),
            pltpu.VMEM((_C,), jnp.int32),
            pltpu.SemaphoreType.DMA,
            pltpu.SemaphoreType.DMA,
        ],
        compiler_params=pltpu.CompilerParams(needs_layout_passes=False),
    )
    def sc_search(a_hbm, b_hbm, c_hbm, fcomb_hbm, shi_hbm, slo_hbm,
                  out_hbm, smp_hi, smp_lo, av, bv, cv, bktv, qhv, qlv,
                  win, outv, sem0, sem1):
        wid = lax.axis_index("s") * _NC + lax.axis_index("c")
        pltpu.sync_copy(shi_hbm, smp_hi)
        pltpu.sync_copy(slo_hbm, smp_lo)
        tile_base = wid * jnp.int32(chunks * _C)
        sems = (sem0, sem1)
        ngrp = _C // _L

        def phase_a(base, buf):
            buf = jnp.int32(buf)
            pltpu.sync_copy(a_hbm.at[pl.ds(base, _C)], av)
            pltpu.sync_copy(b_hbm.at[pl.ds(base, _C)], bv)
            pltpu.sync_copy(c_hbm.at[pl.ds(base, _C)], cv)
            for blk in range(0, ngrp, _BLK):
                keys, pos = [], []
                for g in range(blk, blk + _BLK):
                    a16 = av[pl.ds(g * _L, _L)]
                    b16 = bv[pl.ds(g * _L, _L)]
                    c16 = cv[pl.ds(g * _L, _L)]
                    qh, ql = _pack_limbs(a16, b16, c16)
                    qhv[buf, pl.ds(g * _L, _L)] = qh
                    qlv[buf, pl.ds(g * _L, _L)] = ql
                    keys.append((qh, ql))
                    pos.append(jnp.zeros((_L,), jnp.int32))
                for sstep in range(steps_a - 1, -1, -1):
                    ts, ths, tls = [], [], []
                    for i in range(_BLK):
                        t = pos[i] + (1 << sstep)
                        ts.append(t)
                        ths.append(plsc.load_gather(smp_hi, [t - 1]))
                        tls.append(plsc.load_gather(smp_lo, [t - 1]))
                    for i in range(_BLK):
                        qh, ql = keys[i]
                        less = (ths[i] < qh) | ((ths[i] == qh) & (tls[i] < ql))
                        pos[i] = jnp.where(less, ts[i], pos[i])
                for i in range(_BLK):
                    bktv[buf, pl.ds((blk + i) * _L, _L)] = pos[i]

        def fire(buf, k):
            return pltpu.async_copy(
                fcomb_hbm.at[bktv.at[jnp.int32(buf)]],
                win.at[jnp.int32(buf)], sems[k % 2])

        def phase_b(base, buf):
            buf = jnp.int32(buf)
            for blk in range(0, ngrp, _BLK):
                keys, pos, rows = [], [], []
                for g in range(blk, blk + _BLK):
                    qh = qhv[buf, pl.ds(g * _L, _L)]
                    ql = qlv[buf, pl.ds(g * _L, _L)]
                    keys.append((qh, ql))
                    rows.append(lax.iota(jnp.int32, _L) + (g * _L))
                    pos.append(jnp.zeros((_L,), jnp.int32))
                wref = win.at[buf]
                for sstep in range(5, -1, -1):
                    ts, ths, tls = [], [], []
                    for i in range(_BLK):
                        t = pos[i] + (1 << sstep)
                        ts.append(t)
                        ths.append(plsc.load_gather(wref, [rows[i], t - 1]))
                        tls.append(plsc.load_gather(wref, [rows[i], t + (_W - 1)]))
                    for i in range(_BLK):
                        qh, ql = keys[i]
                        less = (ths[i] < qh) | ((ths[i] == qh) & (tls[i] < ql))
                        pos[i] = jnp.where(less, ts[i], pos[i])
                for i in range(_BLK):
                    qh, ql = keys[i]
                    fh = plsc.load_gather(wref, [rows[i], pos[i]])
                    fl = plsc.load_gather(wref, [rows[i], pos[i] + _W])
                    hit = (fh == qh) & (fl == ql)
                    outv[pl.ds((blk + i) * _L, _L)] = hit.astype(jnp.int32)
            pltpu.sync_copy(outv, out_hbm.at[pl.ds(base, _C)])

        phase_a(tile_base, 0)
        cp = fire(0, 0)

        def pair_body(j, carry):
            k = j * jnp.int32(2)
            base_k = tile_base + k * jnp.int32(_C)
            phase_a(base_k + jnp.int32(_C), 1)
            fire(1, 1)
            pltpu.make_async_copy(fcomb_hbm.at[bktv.at[jnp.int32(0)]],
                                  win.at[jnp.int32(0)], sem0).wait()
            phase_b(base_k, 0)
            phase_a(base_k + jnp.int32(2 * _C), 0)
            fire(0, 0)
            pltpu.make_async_copy(fcomb_hbm.at[bktv.at[jnp.int32(1)]],
                                  win.at[jnp.int32(1)], sem1).wait()
            phase_b(base_k + jnp.int32(_C), 1)
            return carry

        lax.fori_loop(jnp.int32(0), jnp.int32((chunks - 1) // 2), pair_body,
                      jnp.int32(0))
        last = tile_base + jnp.int32((chunks - 1) * _C)
        pltpu.make_async_copy(fcomb_hbm.at[bktv.at[jnp.int32(0)]],
                              win.at[jnp.int32(0)], sem0).wait()
        phase_b(last, 0)

    return sc_search


@jax.jit
def _fact_index(atoms, fact_hashes):
    n = atoms.shape[0]
    f = fact_hashes.shape[0]

    f_pad = max(_W, 1 << (f - 1).bit_length())
    sample = f_pad // _W
    steps_a = sample.bit_length() - 1
    fhi = (fact_hashes >> 31).astype(jnp.int32)
    flo = (fact_hashes & 0x7FFFFFFF).astype(jnp.int32)
    fhi = jnp.concatenate(
        [fhi, jnp.full((f_pad - f,), 0x3FFFFFFF, jnp.int32)])
    flo = jnp.concatenate(
        [flo, jnp.full((f_pad - f,), 0x7FFFFFFF, jnp.int32)])
    fhi2d = fhi.reshape(sample, _W)
    flo2d = flo.reshape(sample, _W)
    fcomb = jnp.concatenate([fhi2d, flo2d], axis=1)
    smp_hi = fhi2d[:, _W - 1]
    smp_lo = flo2d[:, _W - 1]

    per = _NW * _C
    n_chunks = (n + per - 1) // per
    if n_chunks % 2 == 0:
        n_chunks += 1
    n_pad = n_chunks * per
    cols = []
    for j in range(3):
        col = atoms[:, j].astype(jnp.int32)
        cols.append(jnp.concatenate(
            [col, jnp.zeros((n_pad - n,), jnp.int32)]))

    sc_search = _make_sc_search(n_pad, sample, steps_a)
    out = sc_search(cols[0], cols[1], cols[2], fcomb, smp_hi, smp_lo)
    return out[:n] != 0


def kernel(atoms, fact_hashes):
    if atoms.shape[0] == 0 or fact_hashes.shape[0] == 0:
        return jnp.zeros((atoms.shape[0],), dtype=bool)
    return _fact_index(atoms, fact_hashes)
